# Initial kernel scaffold; baseline (speedup 1.0000x reference)
import jax
import jax.numpy as jnp
from jax.experimental import pallas as pl


def kernel(x, edge_index, W1, b1, g1, be1, W2, b2, g2, be2, W3, b3):
    raise NotImplementedError("write your pallas kernel here")


# speedup vs baseline: 7.9356x; 7.9356x over previous
import jax
import jax.numpy as jnp
from jax import lax
from jax.experimental import pallas as pl
from jax.experimental.pallas import tpu as pltpu
from jax.experimental.pallas import tpu_sc as plsc

N = 10000
E = 320000
D = 128

NC = 2
NS = 16
NW = NC * NS
NPAD = 10240
EW = E // NW
CH = 128
G = 80
RPS = NPAD // NS

_mesh = plsc.VectorSubcoreMesh(core_axis_name="c", subcore_axis_name="s")


def _sc_degree_body(z_hbm, ones_hbm, dstp_hbm, cnt_hbm, dst_v, ones_v, acc,
                    sem):
    del sem
    c = lax.axis_index("c")
    s = lax.axis_index("s")
    wid = s * NC + c
    pltpu.sync_copy(dstp_hbm.at[wid], dst_v)
    pltpu.sync_copy(ones_hbm, ones_v)
    pltpu.sync_copy(z_hbm, acc.at[pl.ds(s * RPS, RPS)])
    plsc.subcore_barrier()

    @pl.loop(0, G)
    def _scatter(g):
        pltpu.sync_copy(ones_v, acc.at[dst_v.at[g]], add=True)

    plsc.subcore_barrier()
    pltpu.sync_copy(acc.at[pl.ds(s * RPS, RPS)],
                    cnt_hbm.at[c, pl.ds(s * RPS, RPS)])


_sc_degree = pl.kernel(
    _sc_degree_body,
    out_type=jax.ShapeDtypeStruct((NC, NPAD, D), jnp.float32),
    mesh=_mesh,
    scratch_types=[
        pltpu.VMEM((G, CH), jnp.int32),
        pltpu.VMEM((CH, D), jnp.float32),
        pltpu.VMEM_SHARED((NPAD, D), jnp.float32),
        pltpu.SemaphoreType.DMA,
    ],
)


def _sc_prop_body(z_hbm, u_hbm, eidx_hbm, p_hbm, idx_v, rows, acc, sems):
    c = lax.axis_index("c")
    s = lax.axis_index("s")
    wid = s * NC + c
    pltpu.sync_copy(eidx_hbm.at[wid], idx_v)
    pltpu.sync_copy(z_hbm, acc.at[pl.ds(s * RPS, RPS)])
    plsc.subcore_barrier()

    @pl.loop(0, G)
    def _main(g):
        pltpu.async_copy(u_hbm.at[idx_v.at[0].at[g]], rows.at[0],
                         sems.at[0]).wait()
        pltpu.sync_copy(rows.at[0], acc.at[idx_v.at[1].at[g]], add=True)

    plsc.subcore_barrier()
    pltpu.sync_copy(acc.at[pl.ds(s * RPS, RPS)],
                    p_hbm.at[c, pl.ds(s * RPS, RPS)])


_sc_prop = pl.kernel(
    _sc_prop_body,
    out_type=jax.ShapeDtypeStruct((NC, NPAD, D), jnp.float32),
    mesh=_mesh,
    scratch_types=[
        pltpu.VMEM((2, G, CH), jnp.int32),
        pltpu.VMEM((1, CH, D), jnp.float32),
        pltpu.VMEM_SHARED((NPAD, D), jnp.float32),
        pltpu.SemaphoreType.DMA((1,)),
    ],
)


BN = 1024


def _tc_mm_body(x_ref, w_ref, h2_ref):
    h2_ref[...] = jnp.dot(x_ref[...], w_ref[...],
                          preferred_element_type=jnp.float32)


def _tc_u_body(cnt_ref, h2_ref, dinv_ref, u_ref):
    cnt = cnt_ref[0, :, 0] + cnt_ref[1, :, 0]
    dinv = lax.rsqrt(1.0 + cnt)[:, None]
    dinv_ref[...] = dinv
    u_ref[...] = dinv * h2_ref[...]


def _tc_epi_body(p_ref, u_ref, dinv_ref, b_ref, g_ref, be_ref, w_ref,
                 un_ref):
    dinv = dinv_ref[...]
    t = dinv * (p_ref[0] + p_ref[1] + u_ref[...]) + b_ref[...]
    t = jnp.maximum(t, 0.0)
    mu = jnp.mean(t, axis=-1, keepdims=True)
    var = jnp.mean((t - mu) ** 2, axis=-1, keepdims=True)
    y = (t - mu) * lax.rsqrt(var + 1e-5) * g_ref[...] + be_ref[...]
    un_ref[...] = dinv * jnp.dot(y, w_ref[...],
                                 preferred_element_type=jnp.float32)


def _tc_fin_body(p_ref, u_ref, dinv_ref, b_ref, o_ref):
    o_ref[...] = dinv_ref[...] * (p_ref[0] + p_ref[1] + u_ref[...]) \
        + b_ref[...]


_GRID = NPAD // BN
_spec_rows = pl.BlockSpec((BN, D), lambda i: (i, 0))
_spec_p = pl.BlockSpec((NC, BN, D), lambda i: (0, i, 0))
_spec_cnt = pl.BlockSpec((NC, BN, D), lambda i: (0, i, 0))
_spec_dinv = pl.BlockSpec((BN, 1), lambda i: (i, 0))
_spec_vec = pl.BlockSpec((1, D), lambda i: (0, 0))
_spec_w = pl.BlockSpec((D, D), lambda i: (0, 0))

_shape_rows = jax.ShapeDtypeStruct((NPAD, D), jnp.float32)

_tc_mm = pl.pallas_call(
    _tc_mm_body,
    grid=(_GRID,),
    in_specs=[_spec_rows, _spec_w],
    out_specs=_spec_rows,
    out_shape=_shape_rows,
)

_tc_u = pl.pallas_call(
    _tc_u_body,
    grid=(_GRID,),
    in_specs=[_spec_cnt, _spec_rows],
    out_specs=[_spec_dinv, _spec_rows],
    out_shape=[jax.ShapeDtypeStruct((NPAD, 1), jnp.float32), _shape_rows],
)

_tc_epi = pl.pallas_call(
    _tc_epi_body,
    grid=(_GRID,),
    in_specs=[_spec_p, _spec_rows, _spec_dinv, _spec_vec, _spec_vec,
              _spec_vec, _spec_w],
    out_specs=_spec_rows,
    out_shape=_shape_rows,
)

_tc_fin = pl.pallas_call(
    _tc_fin_body,
    grid=(_GRID,),
    in_specs=[_spec_p, _spec_rows, _spec_dinv, _spec_vec],
    out_specs=_spec_rows,
    out_shape=_shape_rows,
)


def kernel(x, edge_index, W1, b1, g1, be1, W2, b2, g2, be2, W3, b3):
    ei = edge_index.reshape(2, NW, EW)
    pad = jnp.full((2, NW, G * CH - EW), NPAD - 1, jnp.int32)
    eidx = (jnp.concatenate([ei, pad], axis=2)
            .reshape(2, NW, G, CH).transpose(1, 0, 2, 3))
    dstp = eidx[:, 1].reshape(NW, G, CH)
    x_pad = jnp.pad(x, ((0, NPAD - N), (0, 0)))
    b1r, g1r, be1r = b1.reshape(1, D), g1.reshape(1, D), be1.reshape(1, D)
    b2r, g2r, be2r = b2.reshape(1, D), g2.reshape(1, D), be2.reshape(1, D)
    b3r = b3.reshape(1, D)

    zrows = jnp.zeros((RPS, D), jnp.float32)
    ones = jnp.ones((CH, D), jnp.float32)

    cnt = _sc_degree(zrows, ones, dstp)
    h2 = _tc_mm(x_pad, W1)
    dinv, u = _tc_u(cnt, h2)

    p = _sc_prop(zrows, u, eidx)
    u = _tc_epi(p, u, dinv, b1r, g1r, be1r, W2)
    p = _sc_prop(zrows, u, eidx)
    u = _tc_epi(p, u, dinv, b2r, g2r, be2r, W3)
    p = _sc_prop(zrows, u, eidx)
    out = _tc_fin(p, u, dinv, b3r)
    return out[:N]
